# baseline (speedup 1.0000x reference)
import jax
import jax.numpy as jnp
from jax.experimental import pallas as pl
from jax.experimental.pallas import tpu as pltpu

E = 8
K = 2
CIN = 96
INIT = 48
RED = 12
EPS = 1e-5
H = 224
W = 224

R = 32
T = H // R
NX1 = R + 2
L1 = NX1 * W
L2 = R * W
XB = 512
X1B = 32

RT = 32
TR = H // RT


def _routing_kernel(x_ref, w1_ref, s1_ref, b1_ref, w2_ref, s2_ref, b2_ref,
                    idx_ref, val_ref, acc_ref):
    t = pl.program_id(0)
    part = jnp.sum(x_ref[...], axis=(2, 3))

    @pl.when(t == 0)
    def _():
        acc_ref[...] = part

    @pl.when(t != 0)
    def _():
        acc_ref[...] = acc_ref[...] + part

    @pl.when(t == TR - 1)
    def _():
        pooled = acc_ref[...] * (1.0 / (H * W))
        h = jnp.dot(pooled, w1_ref[...], preferred_element_type=jnp.float32)
        h = h * s1_ref[...] + b1_ref[...]
        h = h * jax.nn.sigmoid(h)
        lg = jnp.dot(h, w2_ref[...], preferred_element_type=jnp.float32)
        lg = lg * s2_ref[...] + b2_ref[...]
        m = jnp.max(lg, axis=1, keepdims=True)
        ex = jnp.exp(lg - m)
        p = ex / jnp.sum(ex, axis=1, keepdims=True)
        iota = jax.lax.broadcasted_iota(jnp.int32, (2, E), 1)
        v0 = jnp.max(p, axis=1, keepdims=True)
        i0 = jnp.min(jnp.where(p == v0, iota, E), axis=1, keepdims=True)
        p2 = jnp.where(iota == i0, -1.0, p)
        v1 = jnp.max(p2, axis=1, keepdims=True)
        i1 = jnp.min(jnp.where(p2 == v1, iota, E), axis=1, keepdims=True)
        ssum = v0 + v1 + 1e-6
        idx_ref[...] = jnp.concatenate([i0, i1], axis=1)
        val_ref[...] = jnp.concatenate([v0 / ssum, v1 / ssum], axis=1)


def _shift_p1(a):
    return jnp.concatenate([jnp.zeros((a.shape[0], 1), a.dtype), a[:, :-1]],
                           axis=1)


def _shift_m1(a):
    return jnp.concatenate([a[:, 1:], jnp.zeros((a.shape[0], 1), a.dtype)],
                           axis=1)


def _conv_kernel(idx_ref, val_ref, xa_ref, xb_ref, xc_ref, w9_ref, bp_ref,
                 wd_ref, bc_ref, out_ref, xs_ref, x1_ref, tb_ref, msk_ref):
    b = pl.program_id(0)
    t = pl.program_id(1)
    e0 = idx_ref[b, 0]
    e1 = idx_ref[b, 1]
    v0 = val_ref[b, 0]
    v1 = val_ref[b, 1]

    @pl.when((b == 0) & (t == 0))
    def _():
        lane = jax.lax.broadcasted_iota(jnp.int32, (1, L1), 1)
        u = jnp.remainder(lane, W)
        msk_ref[0:1, :] = jnp.where(u != W - 1, 1.0, 0.0)
        msk_ref[1:2, :] = jnp.where(u != 0, 1.0, 0.0)

    xs_ref[:, XB:XB + R * W] = xa_ref[0]

    @pl.when(t > 0)
    def _():
        xs_ref[:, XB - 2 * W:XB] = xb_ref[0, :, 2 * W:4 * W]

    @pl.when(t == 0)
    def _():
        xs_ref[:, XB - 2 * W:XB] = jnp.zeros((CIN, 2 * W), jnp.float32)

    @pl.when(t < T - 1)
    def _():
        xs_ref[:, XB + R * W:XB + (R + 2) * W] = xc_ref[0, :, 0:2 * W]

    @pl.when(t == T - 1)
    def _():
        xs_ref[:, XB + R * W:XB + (R + 2) * W] = (
            jnp.zeros((CIN, 2 * W), jnp.float32))

    w9 = jnp.concatenate([w9_ref[e0], w9_ref[e1]], axis=1)
    sh1 = jnp.concatenate([bp_ref[e0], bp_ref[e1]], axis=0)
    wd = jnp.concatenate([wd_ref[e0], wd_ref[e1]], axis=0)
    sh2 = jnp.concatenate([bc_ref[e0], bc_ref[e1]], axis=0)

    pdx = []
    for dx in range(3):
        acc = None
        for dy in range(3):
            s_dy = xs_ref[:, XB - 2 * W + dy * W:XB - 2 * W + dy * W + L1]
            term = jnp.dot(w9[dy * 3 + dx], s_dy,
                           preferred_element_type=jnp.float32)
            acc = term if acc is None else acc + term
        pdx.append(acc)
    m223 = msk_ref[0:1, :]
    m0 = msk_ref[1:2, :]
    raw = _shift_p1(pdx[0] * m223) + pdx[1] + _shift_m1(pdx[2] * m0) + sh1
    y1 = raw * jax.nn.sigmoid(raw)
    x1_ref[:, X1B + W:X1B + W + L1] = y1

    @pl.when(t == 0)
    def _():
        x1_ref[:, X1B + W:X1B + 2 * W] = jnp.zeros((2 * INIT, W), jnp.float32)

    @pl.when(t == T - 1)
    def _():
        x1_ref[:, X1B + NX1 * W:X1B + (NX1 + 1) * W] = (
            jnp.zeros((2 * INIT, W), jnp.float32))

    qdx = []
    for dx in range(3):
        acc = None
        for dy in range(3):
            s_dy = x1_ref[:, X1B + (1 + dy) * W:X1B + (1 + dy) * W + L2]
            term = wd[:, dy * 3 + dx][:, None] * s_dy
            acc = term if acc is None else acc + term
        qdx.append(acc)
    y2r = (_shift_p1(qdx[0] * msk_ref[0:1, :L2]) + qdx[1]
           + _shift_m1(qdx[2] * msk_ref[1:2, :L2]) + sh2)
    y2 = y2r * jax.nn.sigmoid(y2r)

    x1t = x1_ref[0:INIT, X1B + 2 * W:X1B + 2 * W + L2]
    x1b = x1_ref[INIT:2 * INIT, X1B + 2 * W:X1B + 2 * W + L2]
    tb_ref[0:INIT, :] = v0 * x1t + v1 * x1b
    tb_ref[INIT:2 * INIT, :] = v0 * y2[0:INIT] + v1 * y2[INIT:2 * INIT]
    for r in range(R):
        out_ref[0, :, r, :] = tb_ref[:, r * W:(r + 1) * W]


def kernel(x, Wr1, g1, b1, Wr2, g2, b2, Wp, gp, bp, Wc, gc, bc):
    B = x.shape[0]
    inv = 1.0 / jnp.sqrt(1.0 + EPS)

    idx, vals = pl.pallas_call(
        _routing_kernel,
        grid=(TR,),
        in_specs=[
            pl.BlockSpec((B, CIN, RT, W), lambda t: (0, 0, t, 0)),
            pl.BlockSpec((CIN, RED), lambda t: (0, 0)),
            pl.BlockSpec((1, RED), lambda t: (0, 0)),
            pl.BlockSpec((1, RED), lambda t: (0, 0)),
            pl.BlockSpec((RED, E), lambda t: (0, 0)),
            pl.BlockSpec((1, E), lambda t: (0, 0)),
            pl.BlockSpec((1, E), lambda t: (0, 0)),
        ],
        out_specs=[
            pl.BlockSpec((B, K), lambda t: (0, 0)),
            pl.BlockSpec((B, K), lambda t: (0, 0)),
        ],
        out_shape=[
            jax.ShapeDtypeStruct((B, K), jnp.int32),
            jax.ShapeDtypeStruct((B, K), jnp.float32),
        ],
        scratch_shapes=[pltpu.VMEM((B, CIN), jnp.float32)],
    )(x, Wr1.T, (g1 * inv)[None, :], b1[None, :],
      Wr2.T, (g2 * inv)[None, :], b2[None, :])

    sp = gp * inv
    w9 = (Wp * sp[:, :, None, None, None]).transpose(0, 3, 4, 1, 2)
    w9 = w9.reshape(E, 9, INIT, CIN)
    sc = gc * inv
    wd9 = (Wc[:, :, 0] * sc[:, :, None, None]).reshape(E, INIT, 9)
    bp3 = bp[:, :, None]
    bc3 = bc[:, :, None]

    xf = x.reshape(B, CIN, H * W)

    out = pl.pallas_call(
        _conv_kernel,
        grid=(B, T),
        in_specs=[
            pl.BlockSpec(memory_space=pltpu.SMEM),
            pl.BlockSpec(memory_space=pltpu.SMEM),
            pl.BlockSpec((1, CIN, R * W), lambda b, t: (b, 0, t)),
            pl.BlockSpec((1, CIN, 4 * W),
                         lambda b, t: (b, 0, jnp.maximum(8 * t - 1, 0))),
            pl.BlockSpec((1, CIN, 4 * W),
                         lambda b, t: (b, 0, jnp.minimum(8 * t + 8, 55))),
            pl.BlockSpec((E, 9, INIT, CIN), lambda b, t: (0, 0, 0, 0)),
            pl.BlockSpec((E, INIT, 1), lambda b, t: (0, 0, 0)),
            pl.BlockSpec((E, INIT, 9), lambda b, t: (0, 0, 0)),
            pl.BlockSpec((E, INIT, 1), lambda b, t: (0, 0, 0)),
        ],
        out_specs=pl.BlockSpec((1, 2 * INIT, R, W), lambda b, t: (b, 0, t, 0)),
        out_shape=jax.ShapeDtypeStruct((B, 2 * INIT, H, W), jnp.float32),
        scratch_shapes=[
            pltpu.VMEM((CIN, XB + (R + 2) * W), jnp.float32),
            pltpu.VMEM((2 * INIT, X1B + (NX1 + 1) * W), jnp.float32),
            pltpu.VMEM((2 * INIT, L2), jnp.float32),
            pltpu.VMEM((8, L1), jnp.float32),
        ],
    )(idx, vals, xf, xf, xf, w9, bp3, wd9, bc3)

    return out


# speedup vs baseline: 1.2471x; 1.2471x over previous
import jax
import jax.numpy as jnp
from jax.experimental import pallas as pl
from jax.experimental.pallas import tpu as pltpu

E = 8
K = 2
CIN = 96
INIT = 48
RED = 12
EPS = 1e-5
H = 224
W = 224

R = 32
T = H // R
ST = 256
NX1 = R + 2
L1 = NX1 * ST
L2 = R * ST

RT = 32
TR = H // RT


def _routing_kernel(x_ref, w1_ref, s1_ref, b1_ref, w2_ref, s2_ref, b2_ref,
                    idx_ref, val_ref, acc_ref):
    t = pl.program_id(0)
    part = jnp.sum(x_ref[...], axis=(2, 3))

    @pl.when(t == 0)
    def _():
        acc_ref[...] = part

    @pl.when(t != 0)
    def _():
        acc_ref[...] = acc_ref[...] + part

    @pl.when(t == TR - 1)
    def _():
        pooled = acc_ref[...] * (1.0 / (H * W))
        h = jnp.dot(pooled, w1_ref[...], preferred_element_type=jnp.float32)
        h = h * s1_ref[...] + b1_ref[...]
        h = h * jax.nn.sigmoid(h)
        lg = jnp.dot(h, w2_ref[...], preferred_element_type=jnp.float32)
        lg = lg * s2_ref[...] + b2_ref[...]
        m = jnp.max(lg, axis=1, keepdims=True)
        ex = jnp.exp(lg - m)
        p = ex / jnp.sum(ex, axis=1, keepdims=True)
        iota = jax.lax.broadcasted_iota(jnp.int32, (2, E), 1)
        v0 = jnp.max(p, axis=1, keepdims=True)
        i0 = jnp.min(jnp.where(p == v0, iota, E), axis=1, keepdims=True)
        p2 = jnp.where(iota == i0, -1.0, p)
        v1 = jnp.max(p2, axis=1, keepdims=True)
        i1 = jnp.min(jnp.where(p2 == v1, iota, E), axis=1, keepdims=True)
        ssum = v0 + v1 + 1e-6
        idx_ref[...] = jnp.concatenate([i0, i1], axis=1)
        val_ref[...] = jnp.concatenate([v0 / ssum, v1 / ssum], axis=1)


def _shift_p1(a):
    return jnp.concatenate([jnp.zeros((a.shape[0], 1), a.dtype), a[:, :-1]],
                           axis=1)


def _shift_m1(a):
    return jnp.concatenate([a[:, 1:], jnp.zeros((a.shape[0], 1), a.dtype)],
                           axis=1)


def _conv_kernel(idx_ref, val_ref, xa_ref, xb_ref, xc_ref, w9_ref, bp_ref,
                 wd_ref, bc_ref, out_ref, xs_ref, x1_ref, tb_ref, msk_ref,
                 w9s_ref, wds_ref, sh1s_ref, sh2s_ref):
    b = pl.program_id(0)
    t = pl.program_id(1)
    v0 = val_ref[b, 0]
    v1 = val_ref[b, 1]

    @pl.when((b == 0) & (t == 0))
    def _():
        q = jnp.bitwise_and(
            jax.lax.broadcasted_iota(jnp.int32, (CIN, xs_ref.shape[1]), 1),
            ST - 1)
        xs_ref[...] = jnp.where(q < W, xs_ref[...], 0.0)
        ql = jnp.bitwise_and(
            jax.lax.broadcasted_iota(jnp.int32, (1, L1), 1), ST - 1)
        msk_ref[...] = jnp.where(ql < W, 1.0, 0.0)

    @pl.when(t == 0)
    def _():
        e0 = idx_ref[b, 0]
        e1 = idx_ref[b, 1]
        w9s_ref[...] = jnp.concatenate([w9_ref[e0], w9_ref[e1]], axis=1)
        wds_ref[...] = jnp.concatenate([wd_ref[e0], wd_ref[e1]], axis=0)
        sh1s_ref[...] = jnp.concatenate([bp_ref[e0], bp_ref[e1]], axis=0)
        sh2s_ref[...] = jnp.concatenate([bc_ref[e0], bc_ref[e1]], axis=0)

    for i in range(R):
        xs_ref[:, (2 + i) * ST:(2 + i) * ST + W] = (
            xa_ref[0, :, i * W:(i + 1) * W])

    @pl.when(t > 0)
    def _():
        xs_ref[:, 0:W] = xb_ref[0, :, 2 * W:3 * W]
        xs_ref[:, ST:ST + W] = xb_ref[0, :, 3 * W:4 * W]

    @pl.when(t == 0)
    def _():
        z = jnp.zeros((CIN, W), jnp.float32)
        xs_ref[:, 0:W] = z
        xs_ref[:, ST:ST + W] = z

    @pl.when(t < T - 1)
    def _():
        xs_ref[:, 34 * ST:34 * ST + W] = xc_ref[0, :, 0:W]
        xs_ref[:, 35 * ST:35 * ST + W] = xc_ref[0, :, W:2 * W]

    @pl.when(t == T - 1)
    def _():
        z = jnp.zeros((CIN, W), jnp.float32)
        xs_ref[:, 34 * ST:34 * ST + W] = z
        xs_ref[:, 35 * ST:35 * ST + W] = z

    pdx = []
    for dx in range(3):
        acc = None
        for dy in range(3):
            s_dy = xs_ref[:, dy * ST:dy * ST + L1]
            term = jnp.dot(w9s_ref[dy * 3 + dx], s_dy,
                           preferred_element_type=jnp.float32)
            acc = term if acc is None else acc + term
        pdx.append(acc)
    raw = _shift_p1(pdx[0]) + pdx[1] + _shift_m1(pdx[2]) + sh1s_ref[...]
    y1 = raw * jax.nn.sigmoid(raw)
    x1_ref[...] = y1 * msk_ref[...]

    @pl.when(t == 0)
    def _():
        x1_ref[:, 0:ST] = jnp.zeros((2 * INIT, ST), jnp.float32)

    @pl.when(t == T - 1)
    def _():
        x1_ref[:, (NX1 - 1) * ST:NX1 * ST] = (
            jnp.zeros((2 * INIT, ST), jnp.float32))

    wd = wds_ref[...]
    qdx = []
    for dx in range(3):
        acc = None
        for dy in range(3):
            s_dy = x1_ref[:, dy * ST:dy * ST + L2]
            term = wd[:, dy * 3 + dx][:, None] * s_dy
            acc = term if acc is None else acc + term
        qdx.append(acc)
    y2r = _shift_p1(qdx[0]) + qdx[1] + _shift_m1(qdx[2]) + sh2s_ref[...]
    y2 = y2r * jax.nn.sigmoid(y2r)

    tb_ref[0:INIT, :] = (v0 * x1_ref[0:INIT, ST:ST + L2]
                         + v1 * x1_ref[INIT:2 * INIT, ST:ST + L2])
    tb_ref[INIT:2 * INIT, :] = v0 * y2[0:INIT] + v1 * y2[INIT:2 * INIT]
    for r in range(R):
        out_ref[0, :, r, :] = tb_ref[:, r * ST:r * ST + W]


def kernel(x, Wr1, g1, b1, Wr2, g2, b2, Wp, gp, bp, Wc, gc, bc):
    B = x.shape[0]
    inv = 1.0 / jnp.sqrt(1.0 + EPS)

    idx, vals = pl.pallas_call(
        _routing_kernel,
        grid=(TR,),
        in_specs=[
            pl.BlockSpec((B, CIN, RT, W), lambda t: (0, 0, t, 0)),
            pl.BlockSpec((CIN, RED), lambda t: (0, 0)),
            pl.BlockSpec((1, RED), lambda t: (0, 0)),
            pl.BlockSpec((1, RED), lambda t: (0, 0)),
            pl.BlockSpec((RED, E), lambda t: (0, 0)),
            pl.BlockSpec((1, E), lambda t: (0, 0)),
            pl.BlockSpec((1, E), lambda t: (0, 0)),
        ],
        out_specs=[
            pl.BlockSpec((B, K), lambda t: (0, 0)),
            pl.BlockSpec((B, K), lambda t: (0, 0)),
        ],
        out_shape=[
            jax.ShapeDtypeStruct((B, K), jnp.int32),
            jax.ShapeDtypeStruct((B, K), jnp.float32),
        ],
        scratch_shapes=[pltpu.VMEM((B, CIN), jnp.float32)],
    )(x, Wr1.T, (g1 * inv)[None, :], b1[None, :],
      Wr2.T, (g2 * inv)[None, :], b2[None, :])

    sp = gp * inv
    w9 = (Wp * sp[:, :, None, None, None]).transpose(0, 3, 4, 1, 2)
    w9 = w9.reshape(E, 9, INIT, CIN)
    sc = gc * inv
    wd9 = (Wc[:, :, 0] * sc[:, :, None, None]).reshape(E, INIT, 9)
    bp3 = bp[:, :, None]
    bc3 = bc[:, :, None]

    xf = x.reshape(B, CIN, H * W)

    out = pl.pallas_call(
        _conv_kernel,
        grid=(B, T),
        in_specs=[
            pl.BlockSpec(memory_space=pltpu.SMEM),
            pl.BlockSpec(memory_space=pltpu.SMEM),
            pl.BlockSpec((1, CIN, R * W), lambda b, t: (b, 0, t)),
            pl.BlockSpec((1, CIN, 4 * W),
                         lambda b, t: (b, 0, jnp.maximum(8 * t - 1, 0))),
            pl.BlockSpec((1, CIN, 4 * W),
                         lambda b, t: (b, 0, jnp.minimum(8 * t + 8, 55))),
            pl.BlockSpec((E, 9, INIT, CIN), lambda b, t: (0, 0, 0, 0)),
            pl.BlockSpec((E, INIT, 1), lambda b, t: (0, 0, 0)),
            pl.BlockSpec((E, INIT, 9), lambda b, t: (0, 0, 0)),
            pl.BlockSpec((E, INIT, 1), lambda b, t: (0, 0, 0)),
        ],
        out_specs=pl.BlockSpec((1, 2 * INIT, R, W), lambda b, t: (b, 0, t, 0)),
        out_shape=jax.ShapeDtypeStruct((B, 2 * INIT, H, W), jnp.float32),
        scratch_shapes=[
            pltpu.VMEM((CIN, 36 * ST), jnp.float32),
            pltpu.VMEM((2 * INIT, L1), jnp.float32),
            pltpu.VMEM((2 * INIT, L2), jnp.float32),
            pltpu.VMEM((1, L1), jnp.float32),
            pltpu.VMEM((9, 2 * INIT, CIN), jnp.float32),
            pltpu.VMEM((2 * INIT, 9), jnp.float32),
            pltpu.VMEM((2 * INIT, 1), jnp.float32),
            pltpu.VMEM((2 * INIT, 1), jnp.float32),
        ],
    )(idx, vals, xf, xf, xf, w9, bp3, wd9, bc3)

    return out


# speedup vs baseline: 1.4886x; 1.1937x over previous
import jax
import jax.numpy as jnp
from jax.experimental import pallas as pl
from jax.experimental.pallas import tpu as pltpu

E = 8
K = 2
CIN = 96
INIT = 48
RED = 12
EPS = 1e-5
H = 224
W = 224

R = 32
T = H // R
ST = 256
NX1 = R + 2
L1 = NX1 * ST
L2 = R * ST

RT = 32
TR = H // RT


def _routing_kernel(x_ref, w1_ref, s1_ref, b1_ref, w2_ref, s2_ref, b2_ref,
                    idx_ref, val_ref, acc_ref):
    t = pl.program_id(0)
    part = jnp.sum(x_ref[...], axis=(2, 3))

    @pl.when(t == 0)
    def _():
        acc_ref[...] = part

    @pl.when(t != 0)
    def _():
        acc_ref[...] = acc_ref[...] + part

    @pl.when(t == TR - 1)
    def _():
        pooled = acc_ref[...] * (1.0 / (H * W))
        h = jnp.dot(pooled, w1_ref[...], preferred_element_type=jnp.float32)
        h = h * s1_ref[...] + b1_ref[...]
        h = h * jax.nn.sigmoid(h)
        lg = jnp.dot(h, w2_ref[...], preferred_element_type=jnp.float32)
        lg = lg * s2_ref[...] + b2_ref[...]
        m = jnp.max(lg, axis=1, keepdims=True)
        ex = jnp.exp(lg - m)
        p = ex / jnp.sum(ex, axis=1, keepdims=True)
        iota = jax.lax.broadcasted_iota(jnp.int32, (2, E), 1)
        v0 = jnp.max(p, axis=1, keepdims=True)
        i0 = jnp.min(jnp.where(p == v0, iota, E), axis=1, keepdims=True)
        p2 = jnp.where(iota == i0, -1.0, p)
        v1 = jnp.max(p2, axis=1, keepdims=True)
        i1 = jnp.min(jnp.where(p2 == v1, iota, E), axis=1, keepdims=True)
        ssum = v0 + v1 + 1e-6
        idx_ref[...] = jnp.concatenate([i0, i1], axis=1)
        val_ref[...] = jnp.concatenate([v0 / ssum, v1 / ssum], axis=1)


def _shift_p1(a):
    return jnp.concatenate([jnp.zeros((a.shape[0], 1), a.dtype), a[:, :-1]],
                           axis=1)


def _shift_m1(a):
    return jnp.concatenate([a[:, 1:], jnp.zeros((a.shape[0], 1), a.dtype)],
                           axis=1)


def _conv_kernel(idx_ref, val_ref, xa_ref, xb_ref, xc_ref, w9_ref, bp_ref,
                 wd_ref, bc_ref, out_ref, xs_ref, x1_ref, tb_ref, msk_ref,
                 w9s_ref, wds_ref, sh1s_ref, sh2s_ref):
    b = pl.program_id(0)
    t = pl.program_id(1)
    v0 = val_ref[b, 0]
    v1 = val_ref[b, 1]

    @pl.when((b == 0) & (t == 0))
    def _():
        q = jnp.bitwise_and(
            jax.lax.broadcasted_iota(jnp.int32, (CIN, xs_ref.shape[1]), 1),
            ST - 1)
        xs_ref[...] = jnp.where(q < W, xs_ref[...], 0.0)
        ql = jnp.bitwise_and(
            jax.lax.broadcasted_iota(jnp.int32, (1, L1), 1), ST - 1)
        msk_ref[...] = jnp.where(ql < W, 1.0, 0.0)

    @pl.when(t == 0)
    def _():
        e0 = idx_ref[b, 0]
        e1 = idx_ref[b, 1]
        w9s_ref[...] = jnp.concatenate([w9_ref[e0], w9_ref[e1]], axis=1)
        wds_ref[...] = jnp.concatenate([wd_ref[e0], wd_ref[e1]], axis=0)
        sh1s_ref[...] = jnp.concatenate([bp_ref[e0], bp_ref[e1]], axis=0)
        sh2s_ref[...] = jnp.concatenate([bc_ref[e0], bc_ref[e1]], axis=0)

    for i in range(R):
        xs_ref[:, (2 + i) * ST:(2 + i) * ST + W] = xa_ref[0, :, i, :]

    @pl.when(t > 0)
    def _():
        xs_ref[:, 0:W] = xb_ref[0, :, 6, :]
        xs_ref[:, ST:ST + W] = xb_ref[0, :, 7, :]

    @pl.when(t == 0)
    def _():
        z = jnp.zeros((CIN, W), jnp.float32)
        xs_ref[:, 0:W] = z
        xs_ref[:, ST:ST + W] = z

    @pl.when(t < T - 1)
    def _():
        xs_ref[:, 34 * ST:34 * ST + W] = xc_ref[0, :, 0, :]
        xs_ref[:, 35 * ST:35 * ST + W] = xc_ref[0, :, 1, :]

    @pl.when(t == T - 1)
    def _():
        z = jnp.zeros((CIN, W), jnp.float32)
        xs_ref[:, 34 * ST:34 * ST + W] = z
        xs_ref[:, 35 * ST:35 * ST + W] = z

    pdx = []
    for dx in range(3):
        acc = None
        for dy in range(3):
            s_dy = xs_ref[:, dy * ST:dy * ST + L1]
            term = jnp.dot(w9s_ref[dy * 3 + dx], s_dy,
                           preferred_element_type=jnp.float32)
            acc = term if acc is None else acc + term
        pdx.append(acc)
    raw = _shift_p1(pdx[0]) + pdx[1] + _shift_m1(pdx[2]) + sh1s_ref[...]
    y1 = raw * jax.nn.sigmoid(raw)
    x1_ref[...] = y1 * msk_ref[...]

    @pl.when(t == 0)
    def _():
        x1_ref[:, 0:ST] = jnp.zeros((2 * INIT, ST), jnp.float32)

    @pl.when(t == T - 1)
    def _():
        x1_ref[:, (NX1 - 1) * ST:NX1 * ST] = (
            jnp.zeros((2 * INIT, ST), jnp.float32))

    wd = wds_ref[...]
    qdx = []
    for dx in range(3):
        acc = None
        for dy in range(3):
            s_dy = x1_ref[:, dy * ST:dy * ST + L2]
            term = wd[:, dy * 3 + dx][:, None] * s_dy
            acc = term if acc is None else acc + term
        qdx.append(acc)
    y2r = _shift_p1(qdx[0]) + qdx[1] + _shift_m1(qdx[2]) + sh2s_ref[...]
    y2 = y2r * jax.nn.sigmoid(y2r)

    tb_ref[0:INIT, :] = (v0 * x1_ref[0:INIT, ST:ST + L2]
                         + v1 * x1_ref[INIT:2 * INIT, ST:ST + L2])
    tb_ref[INIT:2 * INIT, :] = v0 * y2[0:INIT] + v1 * y2[INIT:2 * INIT]
    for r in range(R):
        out_ref[0, :, r, :] = tb_ref[:, r * ST:r * ST + W]


def kernel(x, Wr1, g1, b1, Wr2, g2, b2, Wp, gp, bp, Wc, gc, bc):
    B = x.shape[0]
    inv = 1.0 / jnp.sqrt(1.0 + EPS)

    idx, vals = pl.pallas_call(
        _routing_kernel,
        grid=(TR,),
        in_specs=[
            pl.BlockSpec((B, CIN, RT, W), lambda t: (0, 0, t, 0)),
            pl.BlockSpec((CIN, RED), lambda t: (0, 0)),
            pl.BlockSpec((1, RED), lambda t: (0, 0)),
            pl.BlockSpec((1, RED), lambda t: (0, 0)),
            pl.BlockSpec((RED, E), lambda t: (0, 0)),
            pl.BlockSpec((1, E), lambda t: (0, 0)),
            pl.BlockSpec((1, E), lambda t: (0, 0)),
        ],
        out_specs=[
            pl.BlockSpec((B, K), lambda t: (0, 0)),
            pl.BlockSpec((B, K), lambda t: (0, 0)),
        ],
        out_shape=[
            jax.ShapeDtypeStruct((B, K), jnp.int32),
            jax.ShapeDtypeStruct((B, K), jnp.float32),
        ],
        scratch_shapes=[pltpu.VMEM((B, CIN), jnp.float32)],
    )(x, Wr1.T, (g1 * inv)[None, :], b1[None, :],
      Wr2.T, (g2 * inv)[None, :], b2[None, :])

    sp = gp * inv
    w9 = (Wp * sp[:, :, None, None, None]).transpose(0, 3, 4, 1, 2)
    w9 = w9.reshape(E, 9, INIT, CIN)
    sc = gc * inv
    wd9 = (Wc[:, :, 0] * sc[:, :, None, None]).reshape(E, INIT, 9)
    bp3 = bp[:, :, None]
    bc3 = bc[:, :, None]

    xf = x.reshape(B, CIN, H * W)

    out = pl.pallas_call(
        _conv_kernel,
        grid=(B, T),
        in_specs=[
            pl.BlockSpec(memory_space=pltpu.SMEM),
            pl.BlockSpec(memory_space=pltpu.SMEM),
            pl.BlockSpec((1, CIN, R, W), lambda b, t: (b, 0, t, 0)),
            pl.BlockSpec((1, CIN, 8, W),
                         lambda b, t: (b, 0, jnp.maximum(4 * t - 1, 0), 0)),
            pl.BlockSpec((1, CIN, 8, W),
                         lambda b, t: (b, 0, jnp.minimum(4 * t + 4, 27), 0)),
            pl.BlockSpec((E, 9, INIT, CIN), lambda b, t: (0, 0, 0, 0)),
            pl.BlockSpec((E, INIT, 1), lambda b, t: (0, 0, 0)),
            pl.BlockSpec((E, INIT, 9), lambda b, t: (0, 0, 0)),
            pl.BlockSpec((E, INIT, 1), lambda b, t: (0, 0, 0)),
        ],
        out_specs=pl.BlockSpec((1, 2 * INIT, R, W), lambda b, t: (b, 0, t, 0)),
        out_shape=jax.ShapeDtypeStruct((B, 2 * INIT, H, W), jnp.float32),
        scratch_shapes=[
            pltpu.VMEM((CIN, 36 * ST), jnp.float32),
            pltpu.VMEM((2 * INIT, L1), jnp.float32),
            pltpu.VMEM((2 * INIT, L2), jnp.float32),
            pltpu.VMEM((1, L1), jnp.float32),
            pltpu.VMEM((9, 2 * INIT, CIN), jnp.float32),
            pltpu.VMEM((2 * INIT, 9), jnp.float32),
            pltpu.VMEM((2 * INIT, 1), jnp.float32),
            pltpu.VMEM((2 * INIT, 1), jnp.float32),
        ],
    )(idx, vals, x, x, x, w9, bp3, wd9, bc3)

    return out


# speedup vs baseline: 1.5237x; 1.0236x over previous
import jax
import jax.numpy as jnp
from jax.experimental import pallas as pl
from jax.experimental.pallas import tpu as pltpu

E = 8
K = 2
CIN = 96
INIT = 48
RED = 12
EPS = 1e-5
H = 224
W = 224

R = 32
T = H // R
ST = 256
NX1 = R + 2
L1 = NX1 * ST
L2 = R * ST

RT = 32
TR = H // RT


def _routing_kernel(x_ref, w1_ref, s1_ref, b1_ref, w2_ref, s2_ref, b2_ref,
                    idx_ref, val_ref, acc_ref):
    t = pl.program_id(0)
    part = jnp.sum(x_ref[...], axis=(2, 3))

    @pl.when(t == 0)
    def _():
        acc_ref[...] = part

    @pl.when(t != 0)
    def _():
        acc_ref[...] = acc_ref[...] + part

    @pl.when(t == TR - 1)
    def _():
        pooled = acc_ref[...] * (1.0 / (H * W))
        h = jnp.dot(pooled, w1_ref[...], preferred_element_type=jnp.float32)
        h = h * s1_ref[...] + b1_ref[...]
        h = h * jax.nn.sigmoid(h)
        lg = jnp.dot(h, w2_ref[...], preferred_element_type=jnp.float32)
        lg = lg * s2_ref[...] + b2_ref[...]
        m = jnp.max(lg, axis=1, keepdims=True)
        ex = jnp.exp(lg - m)
        p = ex / jnp.sum(ex, axis=1, keepdims=True)
        iota = jax.lax.broadcasted_iota(jnp.int32, (2, E), 1)
        v0 = jnp.max(p, axis=1, keepdims=True)
        i0 = jnp.min(jnp.where(p == v0, iota, E), axis=1, keepdims=True)
        p2 = jnp.where(iota == i0, -1.0, p)
        v1 = jnp.max(p2, axis=1, keepdims=True)
        i1 = jnp.min(jnp.where(p2 == v1, iota, E), axis=1, keepdims=True)
        ssum = v0 + v1 + 1e-6
        idx_ref[...] = jnp.concatenate([i0, i1], axis=1)
        val_ref[...] = jnp.concatenate([v0 / ssum, v1 / ssum], axis=1)


def _shift_p1(a):
    return jnp.concatenate([jnp.zeros((a.shape[0], 1), a.dtype), a[:, :-1]],
                           axis=1)


def _shift_m1(a):
    return jnp.concatenate([a[:, 1:], jnp.zeros((a.shape[0], 1), a.dtype)],
                           axis=1)


def _conv_kernel(idx_ref, val_ref, xa_ref, xb_ref, xc_ref, w9_ref, bp_ref,
                 wd_ref, bc_ref, out_ref, xs_ref, x1_ref, tb_ref, msk_ref,
                 w9s_ref, wds_ref, sh1s_ref, sh2s_ref):
    b = pl.program_id(0)
    t = pl.program_id(1)
    v0 = val_ref[b, 0]
    v1 = val_ref[b, 1]

    @pl.when((b == 0) & (t == 0))
    def _():
        q = jnp.bitwise_and(
            jax.lax.broadcasted_iota(jnp.int32, (CIN, xs_ref.shape[1]), 1),
            ST - 1)
        xs_ref[...] = jnp.where(q < W, xs_ref[...], 0.0)
        ql = jnp.bitwise_and(
            jax.lax.broadcasted_iota(jnp.int32, (1, L1), 1), ST - 1)
        msk_ref[...] = jnp.where(ql < W, 1.0, 0.0)

    @pl.when(t == 0)
    def _():
        e0 = idx_ref[b, 0]
        e1 = idx_ref[b, 1]
        for dy in range(3):
            blk = []
            for dx in range(3):
                tap = dy * 3 + dx
                blk.append(jnp.concatenate([w9_ref[e0, tap], w9_ref[e1, tap]],
                                           axis=0))
            w9s_ref[dy] = jnp.concatenate(blk, axis=0)
        wds_ref[...] = jnp.concatenate([wd_ref[e0], wd_ref[e1]], axis=0)
        sh1s_ref[...] = jnp.concatenate([bp_ref[e0], bp_ref[e1]], axis=0)
        sh2s_ref[...] = jnp.concatenate([bc_ref[e0], bc_ref[e1]], axis=0)

    for i in range(R):
        xs_ref[:, (2 + i) * ST:(2 + i) * ST + W] = xa_ref[0, :, i, :]

    @pl.when(t > 0)
    def _():
        xs_ref[:, 0:W] = xb_ref[0, :, 6, :]
        xs_ref[:, ST:ST + W] = xb_ref[0, :, 7, :]

    @pl.when(t == 0)
    def _():
        z = jnp.zeros((CIN, W), jnp.float32)
        xs_ref[:, 0:W] = z
        xs_ref[:, ST:ST + W] = z

    @pl.when(t < T - 1)
    def _():
        xs_ref[:, 34 * ST:34 * ST + W] = xc_ref[0, :, 0, :]
        xs_ref[:, 35 * ST:35 * ST + W] = xc_ref[0, :, 1, :]

    @pl.when(t == T - 1)
    def _():
        z = jnp.zeros((CIN, W), jnp.float32)
        xs_ref[:, 34 * ST:34 * ST + W] = z
        xs_ref[:, 35 * ST:35 * ST + W] = z

    pdx = [None, None, None]
    for dy in range(3):
        s_dy = xs_ref[:, dy * ST:dy * ST + L1]
        pall = jnp.dot(w9s_ref[dy], s_dy, preferred_element_type=jnp.float32)
        for dx in range(3):
            sl = pall[dx * 2 * INIT:(dx + 1) * 2 * INIT]
            pdx[dx] = sl if pdx[dx] is None else pdx[dx] + sl
    raw = _shift_p1(pdx[0]) + pdx[1] + _shift_m1(pdx[2]) + sh1s_ref[...]
    y1 = raw * jax.nn.sigmoid(raw)
    x1_ref[...] = y1 * msk_ref[...]

    @pl.when(t == 0)
    def _():
        x1_ref[:, 0:ST] = jnp.zeros((2 * INIT, ST), jnp.float32)

    @pl.when(t == T - 1)
    def _():
        x1_ref[:, (NX1 - 1) * ST:NX1 * ST] = (
            jnp.zeros((2 * INIT, ST), jnp.float32))

    wd = wds_ref[...]
    qdx = []
    for dx in range(3):
        acc = None
        for dy in range(3):
            s_dy = x1_ref[:, dy * ST:dy * ST + L2]
            term = wd[:, dy * 3 + dx][:, None] * s_dy
            acc = term if acc is None else acc + term
        qdx.append(acc)
    y2r = _shift_p1(qdx[0]) + qdx[1] + _shift_m1(qdx[2]) + sh2s_ref[...]
    y2 = y2r * jax.nn.sigmoid(y2r)

    tb_ref[0:INIT, :] = (v0 * x1_ref[0:INIT, ST:ST + L2]
                         + v1 * x1_ref[INIT:2 * INIT, ST:ST + L2])
    tb_ref[INIT:2 * INIT, :] = v0 * y2[0:INIT] + v1 * y2[INIT:2 * INIT]
    for r in range(R):
        out_ref[0, :, r, :] = tb_ref[:, r * ST:r * ST + W]


def kernel(x, Wr1, g1, b1, Wr2, g2, b2, Wp, gp, bp, Wc, gc, bc):
    B = x.shape[0]
    inv = 1.0 / jnp.sqrt(1.0 + EPS)

    idx, vals = pl.pallas_call(
        _routing_kernel,
        grid=(TR,),
        in_specs=[
            pl.BlockSpec((B, CIN, RT, W), lambda t: (0, 0, t, 0)),
            pl.BlockSpec((CIN, RED), lambda t: (0, 0)),
            pl.BlockSpec((1, RED), lambda t: (0, 0)),
            pl.BlockSpec((1, RED), lambda t: (0, 0)),
            pl.BlockSpec((RED, E), lambda t: (0, 0)),
            pl.BlockSpec((1, E), lambda t: (0, 0)),
            pl.BlockSpec((1, E), lambda t: (0, 0)),
        ],
        out_specs=[
            pl.BlockSpec((B, K), lambda t: (0, 0)),
            pl.BlockSpec((B, K), lambda t: (0, 0)),
        ],
        out_shape=[
            jax.ShapeDtypeStruct((B, K), jnp.int32),
            jax.ShapeDtypeStruct((B, K), jnp.float32),
        ],
        scratch_shapes=[pltpu.VMEM((B, CIN), jnp.float32)],
    )(x, Wr1.T, (g1 * inv)[None, :], b1[None, :],
      Wr2.T, (g2 * inv)[None, :], b2[None, :])

    sp = gp * inv
    w9 = (Wp * sp[:, :, None, None, None]).transpose(0, 3, 4, 1, 2)
    w9 = w9.reshape(E, 9, INIT, CIN)
    sc = gc * inv
    wd9 = (Wc[:, :, 0] * sc[:, :, None, None]).reshape(E, INIT, 9)
    bp3 = bp[:, :, None]
    bc3 = bc[:, :, None]

    xf = x.reshape(B, CIN, H * W)

    out = pl.pallas_call(
        _conv_kernel,
        grid=(B, T),
        in_specs=[
            pl.BlockSpec(memory_space=pltpu.SMEM),
            pl.BlockSpec(memory_space=pltpu.SMEM),
            pl.BlockSpec((1, CIN, R, W), lambda b, t: (b, 0, t, 0)),
            pl.BlockSpec((1, CIN, 8, W),
                         lambda b, t: (b, 0, jnp.maximum(4 * t - 1, 0), 0)),
            pl.BlockSpec((1, CIN, 8, W),
                         lambda b, t: (b, 0, jnp.minimum(4 * t + 4, 27), 0)),
            pl.BlockSpec((E, 9, INIT, CIN), lambda b, t: (0, 0, 0, 0)),
            pl.BlockSpec((E, INIT, 1), lambda b, t: (0, 0, 0)),
            pl.BlockSpec((E, INIT, 9), lambda b, t: (0, 0, 0)),
            pl.BlockSpec((E, INIT, 1), lambda b, t: (0, 0, 0)),
        ],
        out_specs=pl.BlockSpec((1, 2 * INIT, R, W), lambda b, t: (b, 0, t, 0)),
        out_shape=jax.ShapeDtypeStruct((B, 2 * INIT, H, W), jnp.float32),
        scratch_shapes=[
            pltpu.VMEM((CIN, 36 * ST), jnp.float32),
            pltpu.VMEM((2 * INIT, L1), jnp.float32),
            pltpu.VMEM((2 * INIT, L2), jnp.float32),
            pltpu.VMEM((1, L1), jnp.float32),
            pltpu.VMEM((3, 6 * INIT, CIN), jnp.float32),
            pltpu.VMEM((2 * INIT, 9), jnp.float32),
            pltpu.VMEM((2 * INIT, 1), jnp.float32),
            pltpu.VMEM((2 * INIT, 1), jnp.float32),
        ],
    )(idx, vals, x, x, x, w9, bp3, wd9, bc3)

    return out


# speedup vs baseline: 1.6034x; 1.0522x over previous
import jax
import jax.numpy as jnp
from jax.experimental import pallas as pl
from jax.experimental.pallas import tpu as pltpu

E = 8
K = 2
CIN = 96
INIT = 48
RED = 12
EPS = 1e-5
H = 224
W = 224

R = 56
T = H // R
ST = 256
NX1 = R + 2
L1 = NX1 * ST
L2 = R * ST

RT = 32
TR = H // RT


def _routing_kernel(x_ref, w1_ref, s1_ref, b1_ref, w2_ref, s2_ref, b2_ref,
                    idx_ref, val_ref, acc_ref):
    t = pl.program_id(0)
    part = jnp.sum(x_ref[...], axis=(2, 3))

    @pl.when(t == 0)
    def _():
        acc_ref[...] = part

    @pl.when(t != 0)
    def _():
        acc_ref[...] = acc_ref[...] + part

    @pl.when(t == TR - 1)
    def _():
        pooled = acc_ref[...] * (1.0 / (H * W))
        h = jnp.dot(pooled, w1_ref[...], preferred_element_type=jnp.float32)
        h = h * s1_ref[...] + b1_ref[...]
        h = h * jax.nn.sigmoid(h)
        lg = jnp.dot(h, w2_ref[...], preferred_element_type=jnp.float32)
        lg = lg * s2_ref[...] + b2_ref[...]
        m = jnp.max(lg, axis=1, keepdims=True)
        ex = jnp.exp(lg - m)
        p = ex / jnp.sum(ex, axis=1, keepdims=True)
        iota = jax.lax.broadcasted_iota(jnp.int32, (2, E), 1)
        v0 = jnp.max(p, axis=1, keepdims=True)
        i0 = jnp.min(jnp.where(p == v0, iota, E), axis=1, keepdims=True)
        p2 = jnp.where(iota == i0, -1.0, p)
        v1 = jnp.max(p2, axis=1, keepdims=True)
        i1 = jnp.min(jnp.where(p2 == v1, iota, E), axis=1, keepdims=True)
        ssum = v0 + v1 + 1e-6
        idx_ref[...] = jnp.concatenate([i0, i1], axis=1)
        val_ref[...] = jnp.concatenate([v0 / ssum, v1 / ssum], axis=1)


def _shift_p1(a):
    return jnp.concatenate([jnp.zeros((a.shape[0], 1), a.dtype), a[:, :-1]],
                           axis=1)


def _shift_m1(a):
    return jnp.concatenate([a[:, 1:], jnp.zeros((a.shape[0], 1), a.dtype)],
                           axis=1)


def _conv_kernel(idx_ref, val_ref, xa_ref, xb_ref, xc_ref, w9_ref, bp_ref,
                 wd_ref, bc_ref, out_ref, xs_ref, x1_ref, tb_ref, msk_ref,
                 w9s_ref, wds_ref, sh1s_ref, sh2s_ref):
    b = pl.program_id(0)
    t = pl.program_id(1)
    v0 = val_ref[b, 0]
    v1 = val_ref[b, 1]

    @pl.when((b == 0) & (t == 0))
    def _():
        q = jnp.bitwise_and(
            jax.lax.broadcasted_iota(jnp.int32, (CIN, xs_ref.shape[1]), 1),
            ST - 1)
        xs_ref[...] = jnp.where(q < W, xs_ref[...], 0.0)
        ql = jnp.bitwise_and(
            jax.lax.broadcasted_iota(jnp.int32, (1, L1), 1), ST - 1)
        msk_ref[...] = jnp.where(ql < W, 1.0, 0.0)

    @pl.when(t == 0)
    def _():
        e0 = idx_ref[b, 0]
        e1 = idx_ref[b, 1]
        for dy in range(3):
            blk = []
            for dx in range(3):
                tap = dy * 3 + dx
                blk.append(jnp.concatenate([w9_ref[e0, tap], w9_ref[e1, tap]],
                                           axis=0))
            w9s_ref[dy] = jnp.concatenate(blk, axis=0)
        wds_ref[...] = jnp.concatenate([wd_ref[e0], wd_ref[e1]], axis=0)
        sh1s_ref[...] = jnp.concatenate([bp_ref[e0], bp_ref[e1]], axis=0)
        sh2s_ref[...] = jnp.concatenate([bc_ref[e0], bc_ref[e1]], axis=0)

    for i in range(R):
        xs_ref[:, (2 + i) * ST:(2 + i) * ST + W] = xa_ref[0, :, i, :]

    @pl.when(t > 0)
    def _():
        xs_ref[:, 0:W] = xb_ref[0, :, 6, :]
        xs_ref[:, ST:ST + W] = xb_ref[0, :, 7, :]

    @pl.when(t == 0)
    def _():
        z = jnp.zeros((CIN, W), jnp.float32)
        xs_ref[:, 0:W] = z
        xs_ref[:, ST:ST + W] = z

    @pl.when(t < T - 1)
    def _():
        xs_ref[:, (R + 2) * ST:(R + 2) * ST + W] = xc_ref[0, :, 0, :]
        xs_ref[:, (R + 3) * ST:(R + 3) * ST + W] = xc_ref[0, :, 1, :]

    @pl.when(t == T - 1)
    def _():
        z = jnp.zeros((CIN, W), jnp.float32)
        xs_ref[:, (R + 2) * ST:(R + 2) * ST + W] = z
        xs_ref[:, (R + 3) * ST:(R + 3) * ST + W] = z

    pdx = [None, None, None]
    for dy in range(3):
        s_dy = xs_ref[:, dy * ST:dy * ST + L1]
        pall = jnp.dot(w9s_ref[dy], s_dy, preferred_element_type=jnp.float32)
        for dx in range(3):
            sl = pall[dx * 2 * INIT:(dx + 1) * 2 * INIT]
            pdx[dx] = sl if pdx[dx] is None else pdx[dx] + sl
    raw = _shift_p1(pdx[0]) + pdx[1] + _shift_m1(pdx[2]) + sh1s_ref[...]
    y1 = raw * jax.nn.sigmoid(raw)
    x1_ref[...] = y1 * msk_ref[...]

    @pl.when(t == 0)
    def _():
        x1_ref[:, 0:ST] = jnp.zeros((2 * INIT, ST), jnp.float32)

    @pl.when(t == T - 1)
    def _():
        x1_ref[:, (NX1 - 1) * ST:NX1 * ST] = (
            jnp.zeros((2 * INIT, ST), jnp.float32))

    wd = wds_ref[...]
    qdx = []
    for dx in range(3):
        acc = None
        for dy in range(3):
            s_dy = x1_ref[:, dy * ST:dy * ST + L2]
            term = wd[:, dy * 3 + dx][:, None] * s_dy
            acc = term if acc is None else acc + term
        qdx.append(acc)
    y2r = _shift_p1(qdx[0]) + qdx[1] + _shift_m1(qdx[2]) + sh2s_ref[...]
    y2 = y2r * jax.nn.sigmoid(y2r)

    tb_ref[0:INIT, :] = (v0 * x1_ref[0:INIT, ST:ST + L2]
                         + v1 * x1_ref[INIT:2 * INIT, ST:ST + L2])
    tb_ref[INIT:2 * INIT, :] = v0 * y2[0:INIT] + v1 * y2[INIT:2 * INIT]
    for r in range(R):
        out_ref[0, :, r, :] = tb_ref[:, r * ST:r * ST + W]


def kernel(x, Wr1, g1, b1, Wr2, g2, b2, Wp, gp, bp, Wc, gc, bc):
    B = x.shape[0]
    inv = 1.0 / jnp.sqrt(1.0 + EPS)

    idx, vals = pl.pallas_call(
        _routing_kernel,
        grid=(TR,),
        in_specs=[
            pl.BlockSpec((B, CIN, RT, W), lambda t: (0, 0, t, 0)),
            pl.BlockSpec((CIN, RED), lambda t: (0, 0)),
            pl.BlockSpec((1, RED), lambda t: (0, 0)),
            pl.BlockSpec((1, RED), lambda t: (0, 0)),
            pl.BlockSpec((RED, E), lambda t: (0, 0)),
            pl.BlockSpec((1, E), lambda t: (0, 0)),
            pl.BlockSpec((1, E), lambda t: (0, 0)),
        ],
        out_specs=[
            pl.BlockSpec((B, K), lambda t: (0, 0)),
            pl.BlockSpec((B, K), lambda t: (0, 0)),
        ],
        out_shape=[
            jax.ShapeDtypeStruct((B, K), jnp.int32),
            jax.ShapeDtypeStruct((B, K), jnp.float32),
        ],
        scratch_shapes=[pltpu.VMEM((B, CIN), jnp.float32)],
    )(x, Wr1.T, (g1 * inv)[None, :], b1[None, :],
      Wr2.T, (g2 * inv)[None, :], b2[None, :])

    sp = gp * inv
    w9 = (Wp * sp[:, :, None, None, None]).transpose(0, 3, 4, 1, 2)
    w9 = w9.reshape(E, 9, INIT, CIN)
    sc = gc * inv
    wd9 = (Wc[:, :, 0] * sc[:, :, None, None]).reshape(E, INIT, 9)
    bp3 = bp[:, :, None]
    bc3 = bc[:, :, None]

    xf = x.reshape(B, CIN, H * W)

    out = pl.pallas_call(
        _conv_kernel,
        grid=(B, T),
        in_specs=[
            pl.BlockSpec(memory_space=pltpu.SMEM),
            pl.BlockSpec(memory_space=pltpu.SMEM),
            pl.BlockSpec((1, CIN, R, W), lambda b, t: (b, 0, t, 0)),
            pl.BlockSpec((1, CIN, 8, W),
                         lambda b, t: (b, 0, jnp.maximum(7 * t - 1, 0), 0)),
            pl.BlockSpec((1, CIN, 8, W),
                         lambda b, t: (b, 0, jnp.minimum(7 * t + 7, 27), 0)),
            pl.BlockSpec((E, 9, INIT, CIN), lambda b, t: (0, 0, 0, 0)),
            pl.BlockSpec((E, INIT, 1), lambda b, t: (0, 0, 0)),
            pl.BlockSpec((E, INIT, 9), lambda b, t: (0, 0, 0)),
            pl.BlockSpec((E, INIT, 1), lambda b, t: (0, 0, 0)),
        ],
        out_specs=pl.BlockSpec((1, 2 * INIT, R, W), lambda b, t: (b, 0, t, 0)),
        out_shape=jax.ShapeDtypeStruct((B, 2 * INIT, H, W), jnp.float32),
        scratch_shapes=[
            pltpu.VMEM((CIN, (R + 4) * ST), jnp.float32),
            pltpu.VMEM((2 * INIT, L1), jnp.float32),
            pltpu.VMEM((2 * INIT, L2), jnp.float32),
            pltpu.VMEM((1, L1), jnp.float32),
            pltpu.VMEM((3, 6 * INIT, CIN), jnp.float32),
            pltpu.VMEM((2 * INIT, 9), jnp.float32),
            pltpu.VMEM((2 * INIT, 1), jnp.float32),
            pltpu.VMEM((2 * INIT, 1), jnp.float32),
        ],
    )(idx, vals, x, x, x, w9, bp3, wd9, bc3)

    return out
